# baseline (speedup 1.0000x reference)
import functools

import jax
import jax.numpy as jnp
from jax import lax
from jax.experimental import pallas as pl
from jax.experimental.pallas import tpu as pltpu
from jax.experimental.pallas import tpu_sc as plsc

DURATION = 100.0
NWORKERS = 32


def _vgather(v, idx):
    return jax.lax.gather(
        v,
        idx[:, None],
        jax.lax.GatherDimensionNumbers(
            offset_dims=(), collapsed_slice_dims=(0,), start_index_map=(0,)
        ),
        (1,),
        mode=jax.lax.GatherScatterMode.PROMISE_IN_BOUNDS,
    )


def _sc_hist_body(off_hbm, out_hbm, buf, cnt_buf):
    nclass = 10
    nper = buf.shape[0] // nclass
    wid = lax.axis_index("s") * 2 + lax.axis_index("c")
    base = wid * buf.shape[0]
    pltpu.sync_copy(off_hbm.at[pl.ds(base, buf.shape[0])], buf)

    lanes = lax.broadcasted_iota(jnp.int32, (16,), 0)
    tenl = lanes * nclass

    def group(g, carry):
        vregs = [buf[pl.ds(g * 160 + r * 16, 16)] for r in range(nclass)]
        vals = []
        for k in range(nclass):
            full = tenl + k
            rvec = full >> 4
            lvec = full & 15
            t = jnp.zeros((16,), jnp.float32)
            for r in range(nclass):
                t = jnp.where(rvec == r, _vgather(vregs[r], lvec), t)
            vals.append(t)
        norm = jnp.abs(vals[0])
        for k in range(1, nclass):
            norm = norm + jnp.abs(vals[k])
        denom = jnp.maximum(norm, 1e-12)
        best = vals[0] / denom
        bidx = jnp.zeros((16,), jnp.int32)
        for k in range(1, nclass):
            p = vals[k] / denom
            m = p > best
            best = jnp.where(m, p, best)
            bidx = jnp.where(m, k, bidx)
        return tuple(
            carry[k] + jnp.where(bidx == k, 1.0, 0.0) for k in range(nclass)
        )

    zero = jnp.zeros((16,), jnp.float32)
    counts = lax.fori_loop(0, nper // 16, group, (zero,) * nclass)
    for k in range(nclass):
        cnt_buf[k, :] = counts[k]
    pltpu.sync_copy(cnt_buf, out_hbm.at[wid])


def _sc_histogram(offsets):
    nneuron, nclass = offsets.shape
    words = (nneuron // NWORKERS) * nclass
    mesh = plsc.VectorSubcoreMesh(core_axis_name="c", subcore_axis_name="s")
    k = functools.partial(
        pl.kernel,
        mesh=mesh,
        out_type=jax.ShapeDtypeStruct((NWORKERS, nclass, 16), jnp.float32),
        scratch_types=[
            pltpu.VMEM((words,), jnp.float32),
            pltpu.VMEM((nclass, 16), jnp.float32),
        ],
    )(_sc_hist_body)
    return k(offsets.reshape(-1))


def _tc_stream_body(x_ref, off_ref, acc_ref):
    i = pl.program_id(0)
    nclass = off_ref.shape[1]

    off = off_ref[...]
    norms = jnp.sum(jnp.abs(off), axis=1, keepdims=True)
    prop = off / jnp.maximum(norms, 1e-12)
    maxv = jnp.max(prop, axis=1, keepdims=True)
    iota = jax.lax.broadcasted_iota(jnp.int32, prop.shape, 1)
    amax = jnp.min(jnp.where(prop == maxv, iota, nclass), axis=1, keepdims=True)
    assoc = jnp.where(iota == amax, prop, 0.0)

    @pl.when(i == 0)
    def _init():
        acc_ref[...] = jnp.zeros_like(acc_ref)

    x = ((DURATION - x_ref[...]) * (1.0 / DURATION)).astype(jnp.bfloat16)
    acc_ref[...] += jnp.dot(
        x, assoc.astype(jnp.bfloat16), preferred_element_type=jnp.float32
    )


def _tc_stream(inputs, offsets):
    batch, nneuron = inputs.shape
    nclass = offsets.shape[1]
    blk_n = 4096
    grid = nneuron // blk_n
    return pl.pallas_call(
        _tc_stream_body,
        grid=(grid,),
        in_specs=[
            pl.BlockSpec((batch, blk_n), lambda i: (0, i)),
            pl.BlockSpec((blk_n, nclass), lambda i: (i, 0)),
        ],
        out_specs=pl.BlockSpec((batch, nclass), lambda i: (0, 0)),
        out_shape=jax.ShapeDtypeStruct((batch, nclass), jnp.float32),
        compiler_params=pltpu.CompilerParams(
            dimension_semantics=("arbitrary",),
        ),
    )(inputs, offsets)


def _tc_finalize_body(acc_ref, parts_ref, out_ref):
    occ = jnp.sum(jnp.sum(parts_ref[...], axis=0), axis=1)
    out_ref[...] = acc_ref[...] / jnp.maximum(occ, 1.0)[None, :]


def _tc_finalize(acc, parts):
    return pl.pallas_call(
        _tc_finalize_body,
        out_shape=jax.ShapeDtypeStruct(acc.shape, jnp.float32),
    )(acc, parts)


def kernel(inputs, offsets):
    occ_parts = _sc_histogram(offsets)
    acc = _tc_stream(inputs, offsets)
    return _tc_finalize(acc, occ_parts)
